# baseline (speedup 1.0000x reference)
import functools

import jax
import jax.numpy as jnp
from jax import lax
from jax.experimental import pallas as pl
from jax.experimental.pallas import tpu as pltpu
from jax.experimental.pallas import tpu_sc as plsc

_SELECTED = (35, 45, 121, 130, 148, 176, 197, 263, 366, 398, 410, 446,
             462, 480, 520, 557, 569, 577, 591, 605, 617, 649, 659, 753)
_NSEL = len(_SELECTED)
_QMAX = (3.0, 15.0, 255.0)

_B, _C, _HW = 32, 768, 1024
_NCORE, _NSUB, _L = 2, 16, 16
_NCH = 32
_CS = _C // _NCH
_NBUF = 4
_NSLICE = _NSEL * (_HW // _L)
_SL_PER_CH = _NSLICE // _NCH


def _round_half_even(y):
    i = y.astype(jnp.int32)
    fi = i.astype(jnp.float32)
    frac = y - fi
    odd = (i & 1) == 1
    up = jnp.logical_or(frac > 0.5,
                        jnp.logical_and(frac == 0.5, odd))
    return fi + jnp.where(up, 1.0, 0.0)


def _sc_body(x_ref, beta_ref, y_ref, ps_ref,
             idx_m, idx_o, gbuf, cbuf, bbuf, statv, sbuf, pbuf, shared,
             gsem, ld0, ld1, ld2, ld3, st0, st1, st2, st3):
    c = lax.axis_index("c")
    s = lax.axis_index("s")
    b_m = 2 * s + c
    b_o = 2 * s + (1 - c)
    lane = lax.iota(jnp.int32, 16)

    def _lanes(vals):
        acc = jnp.zeros((_L,), jnp.int32)
        for k, v in enumerate(vals):
            acc = jnp.where(lane == k, jnp.int32(v), acc)
        return acc

    sel_lo = _lanes(_SELECTED[:16])
    sel_hi = _lanes(_SELECTED[8:24])
    idx_m[pl.ds(0, 16)] = sel_lo + b_m * _C
    idx_m[pl.ds(8, 16)] = sel_hi + b_m * _C
    idx_o[pl.ds(0, 16)] = sel_lo + b_o * _C
    idx_o[pl.ds(8, 16)] = sel_hi + b_o * _C

    ldsems = (ld0, ld1, ld2, ld3)
    stsems = (st0, st1, st2, st3)
    base = b_m * _C

    def load(i):
        return pltpu.make_async_copy(
            x_ref.at[pl.ds(base + i * _CS, _CS)], cbuf.at[i % _NBUF],
            ldsems[i % _NBUF])

    def store(i):
        return pltpu.make_async_copy(
            cbuf.at[i % _NBUF], y_ref.at[pl.ds(base + i * _CS, _CS)],
            stsems[i % _NBUF])

    gath_o = pltpu.make_async_copy(x_ref.at[idx_o], gbuf, gsem)
    gath_o.start()
    for i in range(2):
        load(i).start()

    def stat_body(i, carry):
        r, j, mnv, mxv = carry
        v = gbuf[r, pl.ds(j * _L, _L)]
        mnv = jnp.minimum(mnv, v)
        mxv = jnp.maximum(mxv, v)
        j = j + 1
        wrap = j >= _HW // _L
        return (jnp.where(wrap, r + 1, r),
                jnp.where(wrap, 0, j), mnv, mxv)

    inf_v = jnp.full((_L,), jnp.inf, jnp.float32)
    zero = jnp.int32(0)
    gath_o.wait()
    _, _, mnv, mxv = lax.fori_loop(0, _NSLICE, stat_body,
                                   (zero, zero, inf_v, -inf_v))
    gath_m = pltpu.make_async_copy(x_ref.at[idx_m], gbuf, gsem)
    gath_m.start()
    gath_m.wait()
    _, _, mnv, mxv = lax.fori_loop(0, _NSLICE, stat_body,
                                   (zero, zero, mnv, mxv))
    mn_s = mnv[0]
    mx_s = mxv[0]
    for k in range(1, _L):
        mn_s = jnp.minimum(mn_s, mnv[k])
        mx_s = jnp.maximum(mx_s, mxv[k])
    statv[...] = jnp.where(lane == 0, mn_s,
                           jnp.where(lane == 1, mx_s, 0.0))

    pltpu.sync_copy(statv, shared.at[s])
    plsc.subcore_barrier()
    pltpu.sync_copy(shared, sbuf)
    row0 = sbuf[0]
    mn = row0[0]
    mx = row0[1]
    for i in range(1, _NSUB):
        row = sbuf[i]
        mn = jnp.minimum(mn, row[0])
        mx = jnp.maximum(mx, row[1])

    rngv = jnp.zeros((_L,), jnp.float32) + (mx - mn)
    scales = []
    for qm in _QMAX:
        sc = rngv / qm
        scales.append(jnp.where(sc <= 0.0, jnp.float32(1e-8), sc))
    pltpu.sync_copy(beta_ref, bbuf)
    bv = bbuf[...]
    b0, b1, b2 = bv[0], bv[1], bv[2]
    bmax = jnp.maximum(b0, jnp.maximum(b1, b2))
    evec = jnp.exp(bv - bmax)
    swv = evec / (evec[0] + evec[1] + evec[2])
    sws = [swv[0], swv[1], swv[2]]

    def rw_body(i, carry):
        r, j = carry
        t = gbuf[r, pl.ds(j * _L, _L)] - mn
        acc = jnp.zeros((_L,), jnp.float32)
        for k, qm in enumerate(_QMAX):
            q = jnp.clip(_round_half_even(t / scales[k]), 0.0, qm)
            acc = acc + (q * scales[k] + mn) * sws[k]
        gbuf[r, pl.ds(j * _L, _L)] = acc
        j = j + 1
        wrap = j >= _HW // _L
        return jnp.where(wrap, r + 1, r), jnp.where(wrap, 0, j)

    rw_state = (zero, zero)
    for i in range(_NCH):
        if i + 2 < _NCH:
            if i >= 2:
                store(i - 2).wait()
            load(i + 2).start()
        load(i).wait()
        store(i).start()
        rw_state = lax.fori_loop(0, _SL_PER_CH, rw_body, rw_state)
    for i in range(_NCH - _NBUF, _NCH):
        store(i).wait()

    scat = pltpu.make_async_copy(gbuf, y_ref.at[idx_m], gsem)
    scat.start()
    scat.wait()

    @pl.when(jnp.logical_and(c == 0, s == 0))
    def _():
        pbuf[...] = scales[2]
        pltpu.sync_copy(pbuf, ps_ref)


_sc_call = functools.partial(
    pl.kernel,
    mesh=plsc.VectorSubcoreMesh(core_axis_name="c", subcore_axis_name="s"),
    out_type=[
        jax.ShapeDtypeStruct((_B * _C, _HW), jnp.float32),
        jax.ShapeDtypeStruct((_L,), jnp.float32),
    ],
    scratch_types=[
        pltpu.VMEM((_NSEL,), jnp.int32),
        pltpu.VMEM((_NSEL,), jnp.int32),
        pltpu.VMEM((_NSEL, _HW), jnp.float32),
        pltpu.VMEM((_NBUF, _CS, _HW), jnp.float32),
        pltpu.VMEM((_L,), jnp.float32),
        pltpu.VMEM((_L,), jnp.float32),
        pltpu.VMEM((_NSUB, _L), jnp.float32),
        pltpu.VMEM((_L,), jnp.float32),
        pltpu.VMEM_SHARED((_NSUB, _L), jnp.float32),
    ] + [pltpu.SemaphoreType.DMA] * 9,
)(_sc_body)


def kernel(input, beta_activ, quant_choose):
    del quant_choose
    x = input.reshape(_B * _C, _HW)
    beta16 = jnp.zeros((_L,), jnp.float32).at[:3].set(beta_activ)
    y, ps = _sc_call(x, beta16)
    return y.reshape(input.shape), ps[0]


# speedup vs baseline: 1.1773x; 1.1773x over previous
import functools

import jax
import jax.numpy as jnp
from jax import lax
from jax.experimental import pallas as pl
from jax.experimental.pallas import tpu as pltpu
from jax.experimental.pallas import tpu_sc as plsc

_SELECTED = (35, 45, 121, 130, 148, 176, 197, 263, 366, 398, 410, 446,
             462, 480, 520, 557, 569, 577, 591, 605, 617, 649, 659, 753)
_NSEL = len(_SELECTED)
_QMAX = (3.0, 15.0, 255.0)

_B, _C, _HW = 32, 768, 1024
_L = 16
_KBUF = 8
_DEPTH = 4


def _sc_gather_body(x_ref, slab_ref, idx, gbuf, gsem, osem):
    c = lax.axis_index("c")
    s = lax.axis_index("s")
    b = 2 * s + c
    lane = lax.iota(jnp.int32, _L)

    def _lanes(vals):
        acc = jnp.zeros((_L,), jnp.int32)
        for k, v in enumerate(vals):
            acc = jnp.where(lane == k, jnp.int32(v), acc)
        return acc

    idx[pl.ds(0, 16)] = _lanes(_SELECTED[:16]) + b * _C
    idx[pl.ds(8, 16)] = _lanes(_SELECTED[8:24]) + b * _C

    gath = pltpu.make_async_copy(x_ref.at[idx], gbuf, gsem)
    gath.start()
    gath.wait()
    out = pltpu.make_async_copy(gbuf, slab_ref.at[pl.ds(b * _NSEL, _NSEL)],
                                osem)
    out.start()
    out.wait()


_sc_gather = functools.partial(
    pl.kernel,
    mesh=plsc.VectorSubcoreMesh(core_axis_name="c", subcore_axis_name="s"),
    out_type=jax.ShapeDtypeStruct((_B * _NSEL, _HW), jnp.float32),
    scratch_types=[
        pltpu.VMEM((_NSEL,), jnp.int32),
        pltpu.VMEM((_NSEL, _HW), jnp.float32),
        pltpu.SemaphoreType.DMA,
        pltpu.SemaphoreType.DMA,
    ],
)(_sc_gather_body)


def _transform_rows(buf, gbuf, b, j, p_ref):
    mn = p_ref[0]
    inv0, inv1, inv2 = p_ref[2], p_ref[3], p_ref[4]
    c0, c1, c2 = p_ref[5], p_ref[6], p_ref[7]
    for i, ch in enumerate(_SELECTED):
        t = gbuf[j, i, :] - mn
        acc = c0 * jnp.clip(jnp.round(t * inv0), 0.0, _QMAX[0])
        acc = acc + c1 * jnp.clip(jnp.round(t * inv1), 0.0, _QMAX[1])
        acc = acc + c2 * jnp.clip(jnp.round(t * inv2), 0.0, _QMAX[2])
        buf[b, ch, :] = acc + mn


def _tc_body(x_ref, beta_ref, slab_ref, o_ref, p_ref,
             gbuf, gsem, buf, ld_sems, st_sems):
    def load(j):
        return pltpu.make_async_copy(x_ref.at[j], buf.at[j % _KBUF],
                                     ld_sems.at[j % _KBUF])

    def store(j):
        return pltpu.make_async_copy(buf.at[j % _KBUF], o_ref.at[j],
                                     st_sems.at[j % _KBUF])

    gath = pltpu.make_async_copy(slab_ref, gbuf, gsem)
    gath.start()
    for j in range(_DEPTH):
        load(j).start()

    gath.wait()
    p_ref[0] = jnp.min(gbuf[...])
    p_ref[1] = jnp.max(gbuf[...])
    b0 = beta_ref[0]
    b1 = beta_ref[1]
    b2 = beta_ref[2]
    bmax = jnp.maximum(b0, jnp.maximum(b1, b2))
    e0 = jnp.exp(b0 - bmax)
    e1 = jnp.exp(b1 - bmax)
    e2 = jnp.exp(b2 - bmax)
    tot = e0 + e1 + e2
    sw = (e0 / tot, e1 / tot, e2 / tot)
    rng = p_ref[1] - p_ref[0]
    for i, qm in enumerate(_QMAX):
        s = rng / qm
        s = jnp.where(s <= 0.0, jnp.float32(1e-8), s)
        p_ref[2 + i] = 1.0 / s
        p_ref[5 + i] = sw[i] * s
        if i == len(_QMAX) - 1:
            p_ref[8] = s

    for j in range(_B):
        if j + _DEPTH < _B:
            if j + _DEPTH >= _KBUF:
                store(j + _DEPTH - _KBUF).wait()
            load(j + _DEPTH).start()
        load(j).wait()
        _transform_rows(buf, gbuf, j % _KBUF, j, p_ref)
        store(j).start()
    for j in range(_B - _KBUF, _B):
        store(j).wait()


def kernel(input, beta_activ, quant_choose):
    del quant_choose
    x3 = input.reshape(_B, _C, _HW)
    slab = _sc_gather(input.reshape(_B * _C, _HW))
    slab3 = slab.reshape(_B, _NSEL, _HW)

    out, params = pl.pallas_call(
        _tc_body,
        in_specs=[
            pl.BlockSpec(memory_space=pl.ANY),
            pl.BlockSpec(memory_space=pltpu.SMEM),
            pl.BlockSpec(memory_space=pl.ANY),
        ],
        out_specs=[
            pl.BlockSpec(memory_space=pl.ANY),
            pl.BlockSpec(memory_space=pltpu.SMEM),
        ],
        out_shape=[
            jax.ShapeDtypeStruct((_B, _C, _HW), jnp.float32),
            jax.ShapeDtypeStruct((16,), jnp.float32),
        ],
        scratch_shapes=[
            pltpu.VMEM((_B, _NSEL, _HW), jnp.float32),
            pltpu.SemaphoreType.DMA,
            pltpu.VMEM((_KBUF, _C, _HW), jnp.float32),
            pltpu.SemaphoreType.DMA((_KBUF,)),
            pltpu.SemaphoreType.DMA((_KBUF,)),
        ],
    )(x3, beta_activ, slab3)

    return (out.reshape(input.shape), params[8])
